# baseline (speedup 1.0000x reference)
import functools

import jax
import jax.numpy as jnp
from jax import lax
from jax.experimental import pallas as pl
from jax.experimental.pallas import tpu as pltpu
from jax.experimental.pallas import tpu_sc as plsc

_B = 16384
_D = 33
_NC = 2
_NS = 16
_NW = _NC * _NS
_BPW = _B // _NW
_CHUNK = 128
_NCHUNK = _BPW // _CHUNK
_L = 16
_NGROUP = _BPW // _L
_DP = 48


_LOG2E = 1.4426950408889634
_LN2 = 0.6931471805599453
_EXP_COEFFS = [
    1.0 / 40320.0,
    1.0 / 5040.0,
    1.0 / 720.0,
    1.0 / 120.0,
    1.0 / 24.0,
    1.0 / 6.0,
    0.5,
    1.0,
    1.0,
]


def _exp_precise(x):
    t = x * _LOG2E
    n = t.astype(jnp.int32)
    r = x - n.astype(jnp.float32) * _LN2
    p = jnp.full(r.shape, _EXP_COEFFS[0], jnp.float32)
    for c in _EXP_COEFFS[1:]:
        p = p * r + c
    nc = jnp.maximum(n, jnp.full(n.shape, -126, jnp.int32))
    scale = plsc.bitcast((nc + 127) << 23, jnp.float32)
    return p * scale


def _sc_body(
    ids_hbm, glog_hbm, table_hbm, out_hbm, idx_v, rows_v, outc_v, glog_v, sem
):
    wid = lax.axis_index("s") * _NC + lax.axis_index("c")
    base = wid * _BPW

    pltpu.sync_copy(glog_hbm, glog_v.at[pl.ds(0, _D)])
    for j in range(_NCHUNK):
        pltpu.sync_copy(ids_hbm.at[pl.ds(base + j * _CHUNK, _CHUNK)], idx_v.at[j])

    rows_2d = rows_v
    descs = [
        pltpu.async_copy(
            table_hbm.at[idx_v.at[j]],
            rows_2d.at[pl.ds(j * _CHUNK, _CHUNK)],
            sem,
        )
        for j in range(_NCHUNK)
    ]
    for d in descs:
        d.wait()

    g0 = glog_v[pl.ds(0, _L)]
    g1 = glog_v[pl.ds(_L, _L)]
    g2 = glog_v[pl.ds(2 * _L, _L)]
    glog_s = [g0[j] for j in range(_L)] + [g1[j] for j in range(_L)] + [g2[0]]

    col_idx = [jnp.full((_L,), j, jnp.int32) for j in range(_D)]

    def group_body(g, carry):
        row_idx = g * _L + lax.iota(jnp.int32, 16)
        fidx = [[row_idx, col_idx[j]] for j in range(_D)]
        vals = [
            plsc.load_gather(rows_v, fidx[j]) + glog_s[j] for j in range(_D)
        ]
        m = vals[0]
        for j in range(1, _D):
            m = jnp.maximum(m, vals[j])
        es = [_exp_precise(v - m) for v in vals]
        s = es[0]
        for j in range(1, _D):
            s = s + es[j]
        inv = 1.0 / s
        inv = inv * (2.0 - s * inv)
        for j in range(_D):
            plsc.store_scatter(outc_v, fidx[j], es[j] * inv)
        return carry

    lax.fori_loop(0, _NGROUP, group_body, 0)

    pltpu.sync_copy(outc_v, out_hbm.at[pl.ds(base, _BPW)])


@functools.partial(
    pl.kernel,
    out_type=jax.ShapeDtypeStruct((_B, _D), jnp.float32),
    mesh=plsc.VectorSubcoreMesh(core_axis_name="c", subcore_axis_name="s"),
    scratch_types=[
        pltpu.VMEM((_NCHUNK, _CHUNK), jnp.int32),
        pltpu.VMEM((_BPW, _DP), jnp.float32),
        pltpu.VMEM((_BPW, _D), jnp.float32),
        pltpu.VMEM((3 * _L,), jnp.float32),
        pltpu.SemaphoreType.DMA,
    ],
    compiler_params=pltpu.CompilerParams(
        needs_layout_passes=False, use_tc_tiling_on_sc=False
    ),
)
def _mixer(
    ids_hbm, glog_hbm, table_hbm, out_hbm, idx_v, rows_v, outc_v, glog_v, sem
):
    _sc_body(
        ids_hbm, glog_hbm, table_hbm, out_hbm, idx_v, rows_v, outc_v, glog_v, sem
    )


def kernel(subject_ids, global_logits, subject_bias):
    table = jnp.pad(subject_bias, ((0, 0), (0, _DP - _D)))
    return _mixer(subject_ids, global_logits, table)


# speedup vs baseline: 1.8135x; 1.8135x over previous
import functools

import jax
import jax.numpy as jnp
from jax import lax
from jax.experimental import pallas as pl
from jax.experimental.pallas import tpu as pltpu
from jax.experimental.pallas import tpu_sc as plsc

_B = 16384
_D = 33
_NC = 2
_NS = 16
_NW = _NC * _NS
_BPW = _B // _NW
_L = 16
_NGROUP = _BPW // _L
_DP = 48


def _sc_body(
    ids_hbm, glog_hbm, table_hbm, out_hbm, ids_v, rows_v, outc_v, glog_v, sem
):
    wid = lax.axis_index("s") * _NC + lax.axis_index("c")
    base = wid * _BPW

    pltpu.sync_copy(glog_hbm, glog_v.at[pl.ds(0, _D)])
    pltpu.sync_copy(ids_hbm.at[pl.ds(base, _BPW)], ids_v)

    def enqueue(g, carry):
        vid = ids_v[pl.ds(g * _L, _L)]
        for k in range(_L):
            sid = vid[k]
            pltpu.async_copy(
                table_hbm.at[sid], rows_v.at[g * _L + k, pl.ds(0, _D)], sem
            )
        return carry

    lax.fori_loop(0, _NGROUP, enqueue, 0)

    def drain(r, carry):
        pltpu.make_async_copy(
            table_hbm.at[0], rows_v.at[0, pl.ds(0, _D)], sem
        ).wait()
        return carry

    lax.fori_loop(0, _BPW, drain, 0)

    g0 = glog_v[pl.ds(0, _L)]
    g1 = glog_v[pl.ds(_L, _L)]
    g2 = glog_v[pl.ds(2 * _L, _L)]
    glog_s = [g0[j] for j in range(_L)] + [g1[j] for j in range(_L)] + [g2[0]]

    col_idx = [jnp.full((_L,), j, jnp.int32) for j in range(_D)]

    def group_body(g, carry):
        row_idx = g * _L + lax.iota(jnp.int32, 16)
        fidx = [[row_idx, col_idx[j]] for j in range(_D)]
        obase = row_idx * _D
        vals = [
            plsc.load_gather(rows_v, fidx[j]) + glog_s[j] for j in range(_D)
        ]
        m = vals[0]
        for j in range(1, _D):
            m = jnp.maximum(m, vals[j])
        es = [jnp.exp(v - m) for v in vals]
        s = es[0]
        for j in range(1, _D):
            s = s + es[j]
        inv = 1.0 / s
        for j in range(_D):
            plsc.store_scatter(outc_v, [obase + j], es[j] * inv)
        return carry

    lax.fori_loop(0, _NGROUP, group_body, 0)

    pltpu.sync_copy(outc_v, out_hbm.at[pl.ds(base * _D, _BPW * _D)])


@functools.partial(
    pl.kernel,
    out_type=jax.ShapeDtypeStruct((_B * _D,), jnp.float32),
    mesh=plsc.VectorSubcoreMesh(core_axis_name="c", subcore_axis_name="s"),
    scratch_types=[
        pltpu.VMEM((_BPW,), jnp.int32),
        pltpu.VMEM((_BPW, _DP), jnp.float32),
        pltpu.VMEM((_BPW * _D,), jnp.float32),
        pltpu.VMEM((3 * _L,), jnp.float32),
        pltpu.SemaphoreType.DMA,
    ],
    compiler_params=pltpu.CompilerParams(
        needs_layout_passes=False, use_tc_tiling_on_sc=True
    ),
)
def _mixer(
    ids_hbm, glog_hbm, table_hbm, out_hbm, ids_v, rows_v, outc_v, glog_v, sem
):
    _sc_body(
        ids_hbm, glog_hbm, table_hbm, out_hbm, ids_v, rows_v, outc_v, glog_v, sem
    )


def kernel(subject_ids, global_logits, subject_bias):
    out = _mixer(subject_ids, global_logits, subject_bias)
    return out.reshape(_B, _D)


# speedup vs baseline: 2.3678x; 1.3056x over previous
import functools

import jax
import jax.numpy as jnp
from jax import lax
from jax.experimental import pallas as pl
from jax.experimental.pallas import tpu as pltpu
from jax.experimental.pallas import tpu_sc as plsc

_B = 16384
_D = 33
_NC = 2
_NS = 16
_NW = _NC * _NS
_BPW = _B // _NW
_L = 16
_NGROUP = _BPW // _L
_DP = 48


def _sc_body(
    ids_hbm, glog_hbm, table_hbm, out_hbm, ids_v, rows_v, outt_v, glog_v, sem
):
    wid = lax.axis_index("s") * _NC + lax.axis_index("c")
    base = wid * _BPW

    pltpu.sync_copy(glog_hbm, glog_v.at[pl.ds(0, _D)])
    pltpu.sync_copy(ids_hbm.at[pl.ds(base, _BPW)], ids_v)

    def enqueue(g, carry):
        vid = ids_v[pl.ds(g * _L, _L)]
        for k in range(_L):
            sid = vid[k]
            pltpu.async_copy(
                table_hbm.at[sid], rows_v.at[g * _L + k, pl.ds(0, _D)], sem
            )
        return carry

    lax.fori_loop(0, _NGROUP, enqueue, 0)

    def drain(r, carry):
        pltpu.make_async_copy(
            table_hbm.at[0], rows_v.at[0, pl.ds(0, _D)], sem
        ).wait()
        return carry

    lax.fori_loop(0, _BPW, drain, 0)

    g0 = glog_v[pl.ds(0, _L)]
    g1 = glog_v[pl.ds(_L, _L)]
    g2 = glog_v[pl.ds(2 * _L, _L)]
    glog_s = [g0[j] for j in range(_L)] + [g1[j] for j in range(_L)] + [g2[0]]

    col_idx = [jnp.full((_L,), j, jnp.int32) for j in range(_D)]

    def group_body(g, carry):
        row_idx = g * _L + lax.iota(jnp.int32, 16)
        fidx = [[row_idx, col_idx[j]] for j in range(_D)]
        vals = [
            plsc.load_gather(rows_v, fidx[j]) + glog_s[j] for j in range(_D)
        ]
        m = vals[0]
        for j in range(1, _D):
            m = jnp.maximum(m, vals[j])
        es = [jnp.exp(v - m) for v in vals]
        s = es[0]
        for j in range(1, _D):
            s = s + es[j]
        inv = 1.0 / s
        for j in range(_D):
            outt_v[j, pl.ds(g * _L, _L)] = es[j] * inv
        return carry

    lax.fori_loop(0, _NGROUP, group_body, 0)

    pltpu.sync_copy(outt_v, out_hbm.at[pl.ds(0, _D), pl.ds(base, _BPW)])


@functools.partial(
    pl.kernel,
    out_type=jax.ShapeDtypeStruct((_D, _B), jnp.float32),
    mesh=plsc.VectorSubcoreMesh(core_axis_name="c", subcore_axis_name="s"),
    scratch_types=[
        pltpu.VMEM((_BPW,), jnp.int32),
        pltpu.VMEM((_BPW, _DP), jnp.float32),
        pltpu.VMEM((_D, _BPW), jnp.float32),
        pltpu.VMEM((3 * _L,), jnp.float32),
        pltpu.SemaphoreType.DMA,
    ],
    compiler_params=pltpu.CompilerParams(
        needs_layout_passes=False, use_tc_tiling_on_sc=True
    ),
)
def _mixer(
    ids_hbm, glog_hbm, table_hbm, out_hbm, ids_v, rows_v, outt_v, glog_v, sem
):
    _sc_body(
        ids_hbm, glog_hbm, table_hbm, out_hbm, ids_v, rows_v, outt_v, glog_v, sem
    )


def kernel(subject_ids, global_logits, subject_bias):
    out = _mixer(subject_ids, global_logits, subject_bias)
    return out.T
